# Initial kernel scaffold; baseline (speedup 1.0000x reference)
import jax
import jax.numpy as jnp
from jax.experimental import pallas as pl


def kernel(phoneme_table, language_table, speaker_table, phoneme_ids, language_ids, speaker_ids):
    raise NotImplementedError("write your pallas kernel here")


# speedup vs baseline: 3.5669x; 3.5669x over previous
import dataclasses

import jax
import jax.numpy as jnp
from jax import lax
from jax.experimental import pallas as pl
from jax.experimental.pallas import tpu as pltpu
from jax.experimental.pallas import tpu_sc as plsc

PH_VOCAB, PH_DIM = 1000, 128
LANG_VOCAB, LANG_DIM = 1000, 16
SPK_VOCAB, SPK_DIM = 100000, 64
B, T = 1024, 200
OUT_DIM = PH_DIM + LANG_DIM + SPK_DIM
NC, NS = 2, 16
NW = NC * NS
BPW = B // NW
T0 = 128
T1 = T - T0


def _body(ph_tab, lang_tab, spk_tab, ph_ids, lang_ids, spk_ids, out,
          idx_v, rows_v, sfx_blk, lang_rows, spk_rows, lid_v, sid_v):
    c = lax.axis_index("c")
    s = lax.axis_index("s")
    wid = s * NC + c
    base = wid * BPW

    pltpu.sync_copy(lang_ids.at[pl.ds(base, BPW)], lid_v)
    pltpu.sync_copy(spk_ids.at[pl.ds(base, BPW)], sid_v)

    lanes = lax.broadcasted_iota(jnp.int32, (16,), 0)
    imin = jnp.int32(-2147483648)

    @pl.loop(0, BPW // 16)
    def _(ci):
        lch = lid_v[pl.ds(ci * 16, 16)]
        sch = sid_v[pl.ds(ci * 16, 16)]

        @pl.loop(0, 16)
        def _(j):
            m = lanes == j
            lid = jnp.max(jnp.where(m, lch, imin))
            sid = jnp.max(jnp.where(m, sch, imin))
            i = ci * 16 + j
            pltpu.sync_copy(lang_tab.at[lid], lang_rows.at[i])
            pltpu.sync_copy(spk_tab.at[sid], spk_rows.at[i])

    @pl.loop(0, BPW)
    def _(i):
        b = base + i
        pltpu.sync_copy(ph_ids.at[b], idx_v)
        pltpu.sync_copy(ph_tab.at[idx_v.at[pl.ds(0, T0)]],
                        rows_v.at[pl.ds(0, T0)])
        pltpu.sync_copy(ph_tab.at[idx_v.at[pl.ds(T0, T1)]],
                        rows_v.at[pl.ds(T0, T1)])
        lang_reg = lang_rows[i]
        spk_regs = [spk_rows[i, pl.ds(16 * j, 16)] for j in range(SPK_DIM // 16)]

        @pl.loop(0, T)
        def _(t):
            sfx_blk[t, pl.ds(0, 16)] = lang_reg
            for j, r in enumerate(spk_regs):
                sfx_blk[t, pl.ds(LANG_DIM + 16 * j, 16)] = r
        pltpu.sync_copy(rows_v, out.at[b, :, pl.ds(0, PH_DIM)])
        pltpu.sync_copy(sfx_blk, out.at[b, :, pl.ds(PH_DIM, LANG_DIM + SPK_DIM)])


def kernel(phoneme_table, language_table, speaker_table, phoneme_ids,
           language_ids, speaker_ids):
    mesh = plsc.VectorSubcoreMesh(core_axis_name="c", subcore_axis_name="s")
    cp = pltpu.CompilerParams()
    if "needs_layout_passes" in pltpu.CompilerParams.__dataclass_fields__:
        cp = dataclasses.replace(cp, needs_layout_passes=False)
    f = pl.kernel(
        _body,
        out_type=jax.ShapeDtypeStruct((B, T, OUT_DIM), jnp.float32),
        mesh=mesh,
        compiler_params=cp,
        scratch_types=[
            pltpu.VMEM((T,), jnp.int32),
            pltpu.VMEM((T, PH_DIM), jnp.float32),
            pltpu.VMEM((T, LANG_DIM + SPK_DIM), jnp.float32),
            pltpu.VMEM((BPW, LANG_DIM), jnp.float32),
            pltpu.VMEM((BPW, SPK_DIM), jnp.float32),
            pltpu.VMEM((BPW,), jnp.int32),
            pltpu.VMEM((BPW,), jnp.int32),
        ],
    )
    return f(phoneme_table, language_table, speaker_table,
             phoneme_ids.astype(jnp.int32), language_ids.astype(jnp.int32),
             speaker_ids.astype(jnp.int32))


# speedup vs baseline: 4.3894x; 1.2306x over previous
import dataclasses

import jax
import jax.numpy as jnp
from jax import lax
from jax.experimental import pallas as pl
from jax.experimental.pallas import tpu as pltpu
from jax.experimental.pallas import tpu_sc as plsc

PH_VOCAB, PH_DIM = 1000, 128
LANG_VOCAB, LANG_DIM = 1000, 16
SPK_VOCAB, SPK_DIM = 100000, 64
B, T = 1024, 200
SFX_DIM = LANG_DIM + SPK_DIM
OUT_DIM = PH_DIM + SFX_DIM
NC, NS = 2, 16
NW = NC * NS
BPW = B // NW
T0 = 128
T1 = T - T0


def _body(ph_tab, lang_tab, spk_tab, ph_ids, lang_ids, spk_ids, out,
          idx0, idx1, rows0, rows1, sfx0, sfx1, lang_rows, spk_rows,
          lid_v, sid_v,
          sem_i0, sem_i1, sem_g0, sem_g1, sem_o0, sem_o1, sem_r):
    c = lax.axis_index("c")
    s = lax.axis_index("s")
    wid = s * NC + c
    base = wid * BPW

    idxs = (idx0, idx1)
    rowss = (rows0, rows1)
    sfxs = (sfx0, sfx1)
    sems_i = (sem_i0, sem_i1)
    sems_g = (sem_g0, sem_g1)
    sems_o = (sem_o0, sem_o1)

    pltpu.async_copy(ph_ids.at[base], idx0, sem_i0)

    pltpu.sync_copy(lang_ids.at[pl.ds(base, BPW)], lid_v)
    pltpu.sync_copy(spk_ids.at[pl.ds(base, BPW)], sid_v)

    lanes = lax.broadcasted_iota(jnp.int32, (16,), 0)
    imin = jnp.int32(-2147483648)
    for ci in range(BPW // 16):
        lch = lid_v[pl.ds(ci * 16, 16)]
        sch = sid_v[pl.ds(ci * 16, 16)]
        for j in range(16):
            m = lanes == j
            lid = jnp.max(jnp.where(m, lch, imin))
            sid = jnp.max(jnp.where(m, sch, imin))
            i = ci * 16 + j
            pltpu.async_copy(lang_tab.at[lid], lang_rows.at[i], sem_r)
            pltpu.async_copy(spk_tab.at[sid], spk_rows.at[i], sem_r)
    for i in range(BPW):
        pltpu.make_async_copy(lang_tab.at[0], lang_rows.at[i], sem_r).wait()
        pltpu.make_async_copy(spk_tab.at[0], spk_rows.at[i], sem_r).wait()

    def half(i, p):
        b = base + i

        @pl.when(i >= 2)
        def _():
            pltpu.make_async_copy(
                rowss[p], out.at[b - 2, :, pl.ds(0, PH_DIM)], sems_o[p]).wait()
            pltpu.make_async_copy(
                sfxs[p], out.at[b - 2, :, pl.ds(PH_DIM, SFX_DIM)],
                sems_o[p]).wait()

        pltpu.make_async_copy(ph_ids.at[b], idxs[p], sems_i[p]).wait()
        pltpu.async_copy(ph_tab.at[idxs[p].at[pl.ds(0, T0)]],
                         rowss[p].at[pl.ds(0, T0)], sems_g[p])
        pltpu.async_copy(ph_tab.at[idxs[p].at[pl.ds(T0, T1)]],
                         rowss[p].at[pl.ds(T0, T1)], sems_g[p])

        @pl.when(i + 1 < BPW)
        def _():
            pltpu.async_copy(ph_ids.at[b + 1], idxs[1 - p], sems_i[1 - p])

        sfx = sfxs[p]
        lang_reg = lang_rows[i]
        spk_regs = [spk_rows[i, pl.ds(16 * j, 16)]
                    for j in range(SPK_DIM // 16)]

        @pl.loop(0, T)
        def _(t):
            sfx[t, pl.ds(0, 16)] = lang_reg
            for j, r in enumerate(spk_regs):
                sfx[t, pl.ds(LANG_DIM + 16 * j, 16)] = r

        pltpu.make_async_copy(ph_tab.at[idxs[p].at[pl.ds(0, T0)]],
                              rowss[p].at[pl.ds(0, T0)], sems_g[p]).wait()
        pltpu.make_async_copy(ph_tab.at[idxs[p].at[pl.ds(T0, T1)]],
                              rowss[p].at[pl.ds(T0, T1)], sems_g[p]).wait()
        pltpu.async_copy(rowss[p], out.at[b, :, pl.ds(0, PH_DIM)], sems_o[p])
        pltpu.async_copy(sfx, out.at[b, :, pl.ds(PH_DIM, SFX_DIM)], sems_o[p])

    @pl.loop(0, BPW, step=2)
    def _(i):
        half(i, 0)
        half(i + 1, 1)

    for p in (0, 1):
        b = base + BPW - 2 + p
        pltpu.make_async_copy(
            rowss[p], out.at[b, :, pl.ds(0, PH_DIM)], sems_o[p]).wait()
        pltpu.make_async_copy(
            sfxs[p], out.at[b, :, pl.ds(PH_DIM, SFX_DIM)], sems_o[p]).wait()


def kernel(phoneme_table, language_table, speaker_table, phoneme_ids,
           language_ids, speaker_ids):
    mesh = plsc.VectorSubcoreMesh(core_axis_name="c", subcore_axis_name="s")
    cp = pltpu.CompilerParams()
    if "needs_layout_passes" in pltpu.CompilerParams.__dataclass_fields__:
        cp = dataclasses.replace(cp, needs_layout_passes=False)
    f = pl.kernel(
        _body,
        out_type=jax.ShapeDtypeStruct((B, T, OUT_DIM), jnp.float32),
        mesh=mesh,
        compiler_params=cp,
        scratch_types=[
            pltpu.VMEM((T,), jnp.int32),
            pltpu.VMEM((T,), jnp.int32),
            pltpu.VMEM((T, PH_DIM), jnp.float32),
            pltpu.VMEM((T, PH_DIM), jnp.float32),
            pltpu.VMEM((T, SFX_DIM), jnp.float32),
            pltpu.VMEM((T, SFX_DIM), jnp.float32),
            pltpu.VMEM((BPW, LANG_DIM), jnp.float32),
            pltpu.VMEM((BPW, SPK_DIM), jnp.float32),
            pltpu.VMEM((BPW,), jnp.int32),
            pltpu.VMEM((BPW,), jnp.int32),
            pltpu.SemaphoreType.DMA,
            pltpu.SemaphoreType.DMA,
            pltpu.SemaphoreType.DMA,
            pltpu.SemaphoreType.DMA,
            pltpu.SemaphoreType.DMA,
            pltpu.SemaphoreType.DMA,
            pltpu.SemaphoreType.DMA,
        ],
    )
    return f(phoneme_table, language_table, speaker_table,
             phoneme_ids.astype(jnp.int32), language_ids.astype(jnp.int32),
             speaker_ids.astype(jnp.int32))
